# baseline (speedup 1.0000x reference)
import jax
import jax.numpy as jnp
from jax import lax
from jax.experimental import pallas as pl
from jax.experimental.pallas import tpu as pltpu
from jax.experimental.pallas import tpu_sc as plsc

_B = 4096
_D = 512
_L = 16
_NC = 2
_NS = 16
_NW = _NC * _NS
_RPW = _B // _NW
_CH = 16
_NCHUNK = _RPW // _CH
_MARGIN = 0.05
_SCALE = 1.0 / (2.0 * _B * 4.0)


def _sc_body(l1_hbm, l2_hbm, f1_hbm, f2_hbm, c1_hbm, c2_hbm, out_hbm,
             idx1_v, idx2_v, f1_v, f2_v, c1_v, c2_v,
             t11_v, t22_v, t12_v, t21_v, part_v, sem0, sem1, sem2, sem3):
    c = lax.axis_index("c")
    s = lax.axis_index("s")
    wid = c * _NS + s
    base = wid * _RPW

    sems = (sem0, sem1, sem2)

    def start_feats(ch, buf):
        r0 = base + ch * _CH
        sem = sems[buf]
        return [
            pltpu.async_copy(f1_hbm.at[pl.ds(r0, _CH)], f1_v.at[buf], sem),
            pltpu.async_copy(f2_hbm.at[pl.ds(r0, _CH)], f2_v.at[buf], sem),
        ]

    def start_centers(ch, buf):
        isl = pl.ds(ch * _CH, _CH)
        sem = sems[buf]
        return [
            pltpu.async_copy(c1_hbm.at[idx1_v.at[isl]], c1_v.at[buf], sem),
            pltpu.async_copy(c2_hbm.at[idx2_v.at[isl]], c2_v.at[buf], sem),
        ]

    def start_chunk(ch, buf):
        return start_feats(ch, buf) + start_centers(ch, buf)

    pre0 = start_feats(0, 0)
    pre1 = start_feats(1, 1)
    h1 = pltpu.async_copy(l1_hbm.at[pl.ds(base, _RPW)], idx1_v, sem3)
    h2 = pltpu.async_copy(l2_hbm.at[pl.ds(base, _RPW)], idx2_v, sem3)
    h1.wait()
    h2.wait()

    sub = []
    for s8 in range(2):
        isl = pl.ds(s8 * 8, 8)
        dsl = pl.ds(s8 * 8, 8)
        sub.append([
            pltpu.async_copy(c1_hbm.at[idx1_v.at[isl]], c1_v.at[0, dsl], sem3),
            pltpu.async_copy(c2_hbm.at[idx2_v.at[isl]], c2_v.at[0, dsl], sem3),
        ])

    lanes = lax.iota(jnp.int32, _L)
    col_base = lanes * _L
    zero = jnp.zeros((_L,), jnp.float32)
    acc = zero

    def do_rows(buf, lo, hi):
        f1c, f2c = f1_v.at[buf], f2_v.at[buf]
        c1c, c2c = c1_v.at[buf], c2_v.at[buf]

        def row_body(r, carry):
            s11 = zero
            s22 = zero
            s12 = zero
            s21 = zero
            for j in range(_D // _L):
                sl = pl.ds(j * _L, _L)
                a1 = f1c[r, sl]
                a2 = f2c[r, sl]
                b1 = c1c[r, sl]
                b2 = c2c[r, sl]
                d11 = a1 - b1
                d22 = a2 - b2
                d12 = a1 - b2
                d21 = a2 - b1
                s11 = s11 + d11 * d11
                s22 = s22 + d22 * d22
                s12 = s12 + d12 * d12
                s21 = s21 + d21 * d21
            col = col_base + r
            plsc.store_scatter(t11_v, [col], s11)
            plsc.store_scatter(t22_v, [col], s22)
            plsc.store_scatter(t12_v, [col], s12)
            plsc.store_scatter(t21_v, [col], s21)
            return carry

        lax.fori_loop(lo, hi, row_body, 0)

    def chunk_relu(acc):
        t11 = t11_v[pl.ds(0, _L)]
        t22 = t22_v[pl.ds(0, _L)]
        t12 = t12_v[pl.ds(0, _L)]
        t21 = t21_v[pl.ds(0, _L)]
        for k in range(1, _L):
            sl = pl.ds(k * _L, _L)
            t11 = t11 + t11_v[sl]
            t22 = t22 + t22_v[sl]
            t12 = t12 + t12_v[sl]
            t21 = t21 + t21_v[sl]
        return (acc
                + jnp.maximum(t11 - _MARGIN, 0.0)
                + jnp.maximum(t22 - _MARGIN, 0.0)
                + jnp.maximum(t12 - _MARGIN, 0.0)
                + jnp.maximum(t21 - _MARGIN, 0.0))

    _NBUF = 3
    inflight = [pre1 + start_centers(1, 1), start_chunk(2, 2)]

    for h in pre0:
        h.wait()
    for s8 in range(2):
        for h in sub[s8]:
            h.wait()
        do_rows(0, s8 * 8, s8 * 8 + 8)
    acc = chunk_relu(acc)

    for ch in range(1, _NCHUNK):
        buf = ch % _NBUF
        for h in inflight.pop(0):
            h.wait()
        if ch + 2 < _NCHUNK:
            inflight.append(start_chunk(ch + 2, (ch + 2) % _NBUF))
        do_rows(buf, 0, _CH)
        acc = chunk_relu(acc)

    part_v[...] = acc * _SCALE
    pltpu.sync_copy(part_v, out_hbm.at[wid])


def kernel(label1, label2, feat1, feat2, centers1, centers2):
    mesh = plsc.VectorSubcoreMesh(core_axis_name="c", subcore_axis_name="s")
    run = pl.kernel(
        _sc_body,
        out_type=jax.ShapeDtypeStruct((_NW, _L), jnp.float32),
        mesh=mesh,
        compiler_params=pltpu.CompilerParams(needs_layout_passes=False),
        scratch_types=[
            pltpu.VMEM((_RPW,), jnp.int32),
            pltpu.VMEM((_RPW,), jnp.int32),
            pltpu.VMEM((3, _CH, _D), jnp.float32),
            pltpu.VMEM((3, _CH, _D), jnp.float32),
            pltpu.VMEM((3, _CH, _D), jnp.float32),
            pltpu.VMEM((3, _CH, _D), jnp.float32),
            pltpu.VMEM((_L * _L,), jnp.float32),
            pltpu.VMEM((_L * _L,), jnp.float32),
            pltpu.VMEM((_L * _L,), jnp.float32),
            pltpu.VMEM((_L * _L,), jnp.float32),
            pltpu.VMEM((_L,), jnp.float32),
            pltpu.SemaphoreType.DMA,
            pltpu.SemaphoreType.DMA,
            pltpu.SemaphoreType.DMA,
            pltpu.SemaphoreType.DMA,
        ],
    )
    out = run(label1.astype(jnp.int32), label2.astype(jnp.int32),
              feat1.reshape(_B, _D), feat2.reshape(_B, _D),
              centers1, centers2)
    return jnp.sum(out)


# speedup vs baseline: 1.0202x; 1.0202x over previous
import jax
import jax.numpy as jnp
from jax import lax
from jax.experimental import pallas as pl
from jax.experimental.pallas import tpu as pltpu
from jax.experimental.pallas import tpu_sc as plsc

_B = 4096
_D = 512
_L = 16
_NC = 2
_NS = 16
_NW = _NC * _NS
_RPW = _B // _NW
_CH = 16
_NCHUNK = _RPW // _CH
_MARGIN = 0.05
_SCALE = 1.0 / (2.0 * _B * 4.0)


def _sc_body(l1_hbm, l2_hbm, f1_hbm, f2_hbm, c1_hbm, c2_hbm, out_hbm,
             idx1_v, idx2_v, f1_v, f2_v, c1_v, c2_v,
             t11_v, t22_v, t12_v, t21_v, part_v, sem0, sem1, sem2, sem3):
    c = lax.axis_index("c")
    s = lax.axis_index("s")
    wid = c * _NS + s
    base = wid * _RPW

    sems = (sem0, sem1, sem2)

    def start_feats(ch, buf):
        r0 = base + ch * _CH
        sem = sems[buf]
        return [
            pltpu.async_copy(f1_hbm.at[pl.ds(r0, _CH)], f1_v.at[buf], sem),
            pltpu.async_copy(f2_hbm.at[pl.ds(r0, _CH)], f2_v.at[buf], sem),
        ]

    def start_centers(ch, buf):
        isl = pl.ds(ch * _CH, _CH)
        sem = sems[buf]
        return [
            pltpu.async_copy(c1_hbm.at[idx1_v.at[isl]], c1_v.at[buf], sem),
            pltpu.async_copy(c2_hbm.at[idx2_v.at[isl]], c2_v.at[buf], sem),
        ]

    def start_chunk(ch, buf):
        return start_feats(ch, buf) + start_centers(ch, buf)

    pre0 = start_feats(0, 0)
    pre1 = start_feats(1, 1)
    h1 = pltpu.async_copy(l1_hbm.at[pl.ds(base, _RPW)], idx1_v, sem3)
    h2 = pltpu.async_copy(l2_hbm.at[pl.ds(base, _RPW)], idx2_v, sem3)
    h1.wait()
    h2.wait()


    lanes = lax.iota(jnp.int32, _L)
    col_base = lanes * _L
    zero = jnp.zeros((_L,), jnp.float32)
    acc = zero

    def do_rows(buf, lo, hi):
        f1c, f2c = f1_v.at[buf], f2_v.at[buf]
        c1c, c2c = c1_v.at[buf], c2_v.at[buf]

        def row_body(r, carry):
            s11 = zero
            s22 = zero
            s12 = zero
            s21 = zero
            for j in range(_D // _L):
                sl = pl.ds(j * _L, _L)
                a1 = f1c[r, sl]
                a2 = f2c[r, sl]
                b1 = c1c[r, sl]
                b2 = c2c[r, sl]
                d11 = a1 - b1
                d22 = a2 - b2
                d12 = a1 - b2
                d21 = a2 - b1
                s11 = s11 + d11 * d11
                s22 = s22 + d22 * d22
                s12 = s12 + d12 * d12
                s21 = s21 + d21 * d21
            col = col_base + r
            plsc.store_scatter(t11_v, [col], s11)
            plsc.store_scatter(t22_v, [col], s22)
            plsc.store_scatter(t12_v, [col], s12)
            plsc.store_scatter(t21_v, [col], s21)
            return carry

        lax.fori_loop(lo, hi, row_body, 0)

    def chunk_relu(acc):
        t11 = t11_v[pl.ds(0, _L)]
        t22 = t22_v[pl.ds(0, _L)]
        t12 = t12_v[pl.ds(0, _L)]
        t21 = t21_v[pl.ds(0, _L)]
        for k in range(1, _L):
            sl = pl.ds(k * _L, _L)
            t11 = t11 + t11_v[sl]
            t22 = t22 + t22_v[sl]
            t12 = t12 + t12_v[sl]
            t21 = t21 + t21_v[sl]
        return (acc
                + jnp.maximum(t11 - _MARGIN, 0.0)
                + jnp.maximum(t22 - _MARGIN, 0.0)
                + jnp.maximum(t12 - _MARGIN, 0.0)
                + jnp.maximum(t21 - _MARGIN, 0.0))

    _NBUF = 3
    inflight = [pre0 + start_centers(0, 0), pre1 + start_centers(1, 1)]
    for ch in range(_NCHUNK):
        buf = ch % _NBUF
        for h in inflight.pop(0):
            h.wait()
        if ch + 2 < _NCHUNK:
            inflight.append(start_chunk(ch + 2, (ch + 2) % _NBUF))
        do_rows(buf, 0, _CH)
        acc = chunk_relu(acc)

    part_v[...] = acc * _SCALE
    pltpu.sync_copy(part_v, out_hbm.at[wid])


def kernel(label1, label2, feat1, feat2, centers1, centers2):
    mesh = plsc.VectorSubcoreMesh(core_axis_name="c", subcore_axis_name="s")
    run = pl.kernel(
        _sc_body,
        out_type=jax.ShapeDtypeStruct((_NW, _L), jnp.float32),
        mesh=mesh,
        compiler_params=pltpu.CompilerParams(needs_layout_passes=False),
        scratch_types=[
            pltpu.VMEM((_RPW,), jnp.int32),
            pltpu.VMEM((_RPW,), jnp.int32),
            pltpu.VMEM((3, _CH, _D), jnp.float32),
            pltpu.VMEM((3, _CH, _D), jnp.float32),
            pltpu.VMEM((3, _CH, _D), jnp.float32),
            pltpu.VMEM((3, _CH, _D), jnp.float32),
            pltpu.VMEM((_L * _L,), jnp.float32),
            pltpu.VMEM((_L * _L,), jnp.float32),
            pltpu.VMEM((_L * _L,), jnp.float32),
            pltpu.VMEM((_L * _L,), jnp.float32),
            pltpu.VMEM((_L,), jnp.float32),
            pltpu.SemaphoreType.DMA,
            pltpu.SemaphoreType.DMA,
            pltpu.SemaphoreType.DMA,
            pltpu.SemaphoreType.DMA,
        ],
    )
    out = run(label1.astype(jnp.int32), label2.astype(jnp.int32),
              feat1.reshape(_B, _D), feat2.reshape(_B, _D),
              centers1, centers2)
    return jnp.sum(out)
